# baseline (speedup 1.0000x reference)
import jax
import jax.numpy as jnp
from jax.experimental import pallas as pl
from jax.experimental.pallas import tpu as pltpu

N = 4096
C = 128
IN = 256
HOPS = 4
TH = 0.6
T = 0.07

BI = 512
BJ = 512
GI = N // BI
GJ = N // BJ


def _embed_body(x_ref, w_ref, b_ref, o_ref):
    y = jnp.dot(x_ref[...], w_ref[...], preferred_element_type=jnp.float32)
    y = y + b_ref[...]
    nrm = jnp.sqrt(jnp.sum(y * y, axis=1, keepdims=True))
    o_ref[...] = y / nrm


def _adj_body(ei_ref, ej_ref, a_ref, norm_ref):
    i = pl.program_id(0)
    j = pl.program_id(1)
    d = jax.lax.dot_general(ei_ref[...], ej_ref[...], (((1,), (1,)), ((), ())),
                            preferred_element_type=jnp.float32)
    rowid = jax.lax.broadcasted_iota(jnp.int32, (BI, BJ), 0)
    colid = jax.lax.broadcasted_iota(jnp.int32, (BI, BJ), 1)
    eye = (rowid == colid) & (i == j)
    a = (d > TH).astype(jnp.float32) + eye.astype(jnp.float32)
    a_ref[...] = a.astype(jnp.bfloat16)
    deg = jnp.sum(a, axis=1, keepdims=True)

    @pl.when(j == 0)
    def _():
        norm_ref[...] = deg

    @pl.when(j > 0)
    def _():
        norm_ref[...] += deg

    @pl.when(j == GJ - 1)
    def _():
        norm_ref[...] = jax.lax.rsqrt(jnp.clip(norm_ref[...], 1.0, None))


def _hop_body(a_ref, h_ref, ni_ref, nj_ref, o_ref):
    j = pl.program_id(1)
    hn = (h_ref[...] * nj_ref[...]).astype(jnp.bfloat16)
    p = jnp.dot(a_ref[...], hn, preferred_element_type=jnp.float32)

    @pl.when(j == 0)
    def _():
        o_ref[...] = p

    @pl.when(j > 0)
    def _():
        o_ref[...] += p

    @pl.when(j == GJ - 1)
    def _():
        o_ref[...] *= ni_ref[...]


def _tag_body(h0_ref, h1_ref, h2_ref, h3_ref, h4_ref, w_ref, b_ref,
              gt_ref, gs_ref):
    w = w_ref[...]
    b = b_ref[...]
    hs = (h0_ref[...], h1_ref[...], h2_ref[...], h3_ref[...], h4_ref[...])
    for col, out in ((0, gt_ref), (C, gs_ref)):
        raw = b
        for k in range(HOPS + 1):
            raw = raw + jnp.dot(hs[k][:, col:col + C], w[k * C:(k + 1) * C, :],
                                preferred_element_type=jnp.float32)
        nrm = jnp.sqrt(jnp.sum(raw * raw, axis=1, keepdims=True))
        out[...] = raw / nrm


def _loss_body(gt_ref, gs_ref, o_ref, m_ref, s_ref, p_ref):
    i = pl.program_id(0)
    j = pl.program_id(1)
    tile = jax.lax.dot_general(gt_ref[...].astype(jnp.bfloat16),
                               gs_ref[...].astype(jnp.bfloat16),
                               (((1,), (1,)), ((), ())),
                               preferred_element_type=jnp.float32) * (1.0 / T)

    @pl.when(i == j)
    def _():
        p_ref[...] = jnp.sum(gt_ref[...] * gs_ref[...], axis=1,
                             keepdims=True) * (1.0 / T)

    rowid = jax.lax.broadcasted_iota(jnp.int32, (BI, BJ), 0)
    colid = jax.lax.broadcasted_iota(jnp.int32, (BI, BJ), 1)
    diag = (rowid == colid) & (i == j)
    tile = jnp.where(diag, -10.0 / T, tile)

    @pl.when(j == 0)
    def _():
        m_ref[...] = jnp.full((BI, 1), -jnp.inf, jnp.float32)
        s_ref[...] = jnp.zeros((BI, 1), jnp.float32)

    m_prev = m_ref[...]
    s_prev = s_ref[...]
    m_new = jnp.maximum(m_prev, jnp.max(tile, axis=1, keepdims=True))
    s_new = s_prev * jnp.exp(m_prev - m_new) + jnp.sum(
        jnp.exp(tile - m_new), axis=1, keepdims=True)
    m_ref[...] = m_new
    s_ref[...] = s_new

    @pl.when((i == 0) & (j == 0))
    def _():
        o_ref[...] = jnp.zeros((1, 1), jnp.float32)

    @pl.when(j == GJ - 1)
    def _():
        pos = p_ref[...]
        mm = jnp.maximum(m_ref[...], pos)
        lse = mm + jnp.log(s_ref[...] * jnp.exp(m_ref[...] - mm)
                           + jnp.exp(pos - mm))
        contrib = jnp.sum(lse - pos)
        tot = o_ref[...] + contrib
        o_ref[...] = jnp.where(i == GI - 1, tot * (1.0 / N), tot)


def kernel(feat_s, feat_t, W_embed, b_embed, W_tag, b_tag):
    b_embed2 = b_embed.reshape(1, C)
    b_tag2 = b_tag.reshape(1, C)

    feats = jnp.concatenate([feat_t, feat_s], axis=0)
    emb = pl.pallas_call(
        _embed_body,
        grid=(2 * GI,),
        in_specs=[
            pl.BlockSpec((BI, IN), lambda i: (i, 0)),
            pl.BlockSpec((IN, C), lambda i: (0, 0)),
            pl.BlockSpec((1, C), lambda i: (0, 0)),
        ],
        out_specs=pl.BlockSpec((BI, C), lambda i: (i, 0)),
        out_shape=jax.ShapeDtypeStruct((2 * N, C), jnp.float32),
    )(feats, W_embed, b_embed2)
    f_et = emb[:N]
    f_es = emb[N:]

    adj, norm = pl.pallas_call(
        _adj_body,
        grid=(GI, GJ),
        in_specs=[
            pl.BlockSpec((BI, C), lambda i, j: (i, 0)),
            pl.BlockSpec((BJ, C), lambda i, j: (j, 0)),
        ],
        out_specs=[
            pl.BlockSpec((BI, BJ), lambda i, j: (i, j)),
            pl.BlockSpec((BI, 1), lambda i, j: (i, 0)),
        ],
        out_shape=[
            jax.ShapeDtypeStruct((N, N), jnp.bfloat16),
            jax.ShapeDtypeStruct((N, 1), jnp.float32),
        ],
    )(f_et, f_et)

    hop_call = pl.pallas_call(
        _hop_body,
        grid=(GI, GJ),
        in_specs=[
            pl.BlockSpec((BI, BJ), lambda i, j: (i, j)),
            pl.BlockSpec((BJ, 2 * C), lambda i, j: (j, 0)),
            pl.BlockSpec((BI, 1), lambda i, j: (i, 0)),
            pl.BlockSpec((BJ, 1), lambda i, j: (j, 0)),
        ],
        out_specs=pl.BlockSpec((BI, 2 * C), lambda i, j: (i, 0)),
        out_shape=jax.ShapeDtypeStruct((N, 2 * C), jnp.float32),
    )

    hs = [jnp.concatenate([f_et, f_es], axis=1)]
    for _ in range(HOPS):
        hs.append(hop_call(adj, hs[-1], norm, norm))

    f_gt, f_gs = pl.pallas_call(
        _tag_body,
        grid=(GI,),
        in_specs=[pl.BlockSpec((BI, 2 * C), lambda i: (i, 0))] * (HOPS + 1)
        + [
            pl.BlockSpec(((HOPS + 1) * C, C), lambda i: (0, 0)),
            pl.BlockSpec((1, C), lambda i: (0, 0)),
        ],
        out_specs=[
            pl.BlockSpec((BI, C), lambda i: (i, 0)),
            pl.BlockSpec((BI, C), lambda i: (i, 0)),
        ],
        out_shape=[
            jax.ShapeDtypeStruct((N, C), jnp.float32),
            jax.ShapeDtypeStruct((N, C), jnp.float32),
        ],
    )(*hs, W_tag, b_tag2)

    loss = pl.pallas_call(
        _loss_body,
        grid=(GI, GJ),
        in_specs=[
            pl.BlockSpec((BI, C), lambda i, j: (i, 0)),
            pl.BlockSpec((BJ, C), lambda i, j: (j, 0)),
        ],
        out_specs=pl.BlockSpec((1, 1), lambda i, j: (0, 0)),
        out_shape=jax.ShapeDtypeStruct((1, 1), jnp.float32),
        scratch_shapes=[
            pltpu.VMEM((BI, 1), jnp.float32),
            pltpu.VMEM((BI, 1), jnp.float32),
            pltpu.VMEM((BI, 1), jnp.float32),
        ],
    )(f_gt, f_gs)

    return loss.reshape(())


# speedup vs baseline: 1.3643x; 1.3643x over previous
import jax
import jax.numpy as jnp
from jax.experimental import pallas as pl
from jax.experimental.pallas import tpu as pltpu

N = 4096
C = 128
IN = 256
HOPS = 4
TH = 0.6
T = 0.07

BI = 512
BJ = 512
GI = N // BI
GJ = N // BJ

BJH = 1024
GJH = N // BJH


def _embed_body(x_ref, w_ref, b_ref, o_ref):
    y = jnp.dot(x_ref[...], w_ref[...], preferred_element_type=jnp.float32)
    y = y + b_ref[...]
    nrm = jnp.sqrt(jnp.sum(y * y, axis=1, keepdims=True))
    o_ref[...] = y / nrm


def _adj_body(ei_ref, ej_ref, h0_ref, a_ref, norm_ref, x0_ref):
    i = pl.program_id(0)
    j = pl.program_id(1)
    d = jax.lax.dot_general(ei_ref[...].astype(jnp.bfloat16),
                            ej_ref[...].astype(jnp.bfloat16),
                            (((1,), (1,)), ((), ())),
                            preferred_element_type=jnp.float32)
    base = (d > TH).astype(jnp.float32)

    @pl.when(i == j)
    def _():
        rowid = jax.lax.broadcasted_iota(jnp.int32, (BI, BJ), 0)
        colid = jax.lax.broadcasted_iota(jnp.int32, (BI, BJ), 1)
        a_ref[...] = (base + (rowid == colid).astype(jnp.float32)
                      ).astype(jnp.bfloat16)

    @pl.when(i != j)
    def _():
        a_ref[...] = base.astype(jnp.bfloat16)

    deg = jnp.sum(base, axis=1, keepdims=True)

    @pl.when(j == 0)
    def _():
        norm_ref[...] = deg

    @pl.when(j > 0)
    def _():
        norm_ref[...] += deg

    @pl.when(j == GJ - 1)
    def _():
        nrm = jax.lax.rsqrt(jnp.clip(norm_ref[...] + 1.0, 1.0, None))
        norm_ref[...] = nrm
        x0_ref[...] = (h0_ref[...] * nrm).astype(jnp.bfloat16)


def _hop_body(a_ref, x_ref, ni_ref, h_ref, xn_ref):
    j = pl.program_id(1)
    p = jnp.dot(a_ref[...], x_ref[...], preferred_element_type=jnp.float32)

    @pl.when(j == 0)
    def _():
        h_ref[...] = p

    @pl.when(j > 0)
    def _():
        h_ref[...] += p

    @pl.when(j == GJH - 1)
    def _():
        ni = ni_ref[...]
        hk = h_ref[...] * ni
        h_ref[...] = hk
        xn_ref[...] = (hk * ni).astype(jnp.bfloat16)


def _tag_body(h0_ref, h1_ref, h2_ref, h3_ref, h4_ref, w_ref, b_ref,
              gt_ref, gs_ref):
    w = w_ref[...]
    b = b_ref[...]
    hs = (h0_ref[...], h1_ref[...], h2_ref[...], h3_ref[...], h4_ref[...])
    for col, out, scale in ((0, gt_ref, T), (C, gs_ref, 1.0)):
        raw = b
        for k in range(HOPS + 1):
            raw = raw + jnp.dot(hs[k][:, col:col + C], w[k * C:(k + 1) * C, :],
                                preferred_element_type=jnp.float32)
        nrm = jnp.sqrt(jnp.sum(raw * raw, axis=1, keepdims=True))
        out[...] = (raw / (nrm * scale)).astype(jnp.bfloat16)


def _loss_body(gt_ref, gs_ref, o_ref, s_ref, p_ref):
    i = pl.program_id(0)
    j = pl.program_id(1)
    tile = jax.lax.dot_general(gt_ref[...], gs_ref[...],
                               (((1,), (1,)), ((), ())),
                               preferred_element_type=jnp.float32)

    @pl.when(i == j)
    def _():
        p_ref[...] = jnp.sum(tile * (jax.lax.broadcasted_iota(
            jnp.int32, (BI, BJ), 0) == jax.lax.broadcasted_iota(
            jnp.int32, (BI, BJ), 1)).astype(jnp.float32),
            axis=1, keepdims=True)

    e = jnp.exp(tile)

    @pl.when(i == j)
    def _():
        rowid = jax.lax.broadcasted_iota(jnp.int32, (BI, BJ), 0)
        colid = jax.lax.broadcasted_iota(jnp.int32, (BI, BJ), 1)
        ez = jnp.where(rowid == colid, 0.0, e)
        part = jnp.sum(ez, axis=1, keepdims=True)

        @pl.when(j == 0)
        def _():
            s_ref[...] = part

        @pl.when(j > 0)
        def _():
            s_ref[...] += part

    @pl.when(i != j)
    def _():
        part = jnp.sum(e, axis=1, keepdims=True)

        @pl.when(j == 0)
        def _():
            s_ref[...] = part

        @pl.when(j > 0)
        def _():
            s_ref[...] += part

    @pl.when((i == 0) & (j == 0))
    def _():
        o_ref[...] = jnp.zeros((1, 1), jnp.float32)

    @pl.when(j == GJ - 1)
    def _():
        pos = p_ref[...]
        lse = jnp.log(s_ref[...] + jnp.exp(pos))
        contrib = jnp.sum(lse - pos)
        tot = o_ref[...] + contrib
        o_ref[...] = jnp.where(i == GI - 1, tot * (1.0 / N), tot)


def kernel(feat_s, feat_t, W_embed, b_embed, W_tag, b_tag):
    b_embed2 = b_embed.reshape(1, C)
    b_tag2 = b_tag.reshape(1, C)

    feats = jnp.concatenate([feat_t, feat_s], axis=0)
    emb = pl.pallas_call(
        _embed_body,
        grid=(2 * GI,),
        in_specs=[
            pl.BlockSpec((BI, IN), lambda i: (i, 0)),
            pl.BlockSpec((IN, C), lambda i: (0, 0)),
            pl.BlockSpec((1, C), lambda i: (0, 0)),
        ],
        out_specs=pl.BlockSpec((BI, C), lambda i: (i, 0)),
        out_shape=jax.ShapeDtypeStruct((2 * N, C), jnp.float32),
    )(feats, W_embed, b_embed2)
    f_et = emb[:N]
    f_es = emb[N:]
    h0 = jnp.concatenate([f_et, f_es], axis=1)

    adj, norm, x0 = pl.pallas_call(
        _adj_body,
        grid=(GI, GJ),
        in_specs=[
            pl.BlockSpec((BI, C), lambda i, j: (i, 0)),
            pl.BlockSpec((BJ, C), lambda i, j: (j, 0)),
            pl.BlockSpec((BI, 2 * C), lambda i, j: (i, 0)),
        ],
        out_specs=[
            pl.BlockSpec((BI, BJ), lambda i, j: (i, j)),
            pl.BlockSpec((BI, 1), lambda i, j: (i, 0)),
            pl.BlockSpec((BI, 2 * C), lambda i, j: (i, 0)),
        ],
        out_shape=[
            jax.ShapeDtypeStruct((N, N), jnp.bfloat16),
            jax.ShapeDtypeStruct((N, 1), jnp.float32),
            jax.ShapeDtypeStruct((N, 2 * C), jnp.bfloat16),
        ],
    )(f_et, f_et, h0)

    hop_call = pl.pallas_call(
        _hop_body,
        grid=(GI, GJH),
        in_specs=[
            pl.BlockSpec((BI, BJH), lambda i, j: (i, j)),
            pl.BlockSpec((BJH, 2 * C), lambda i, j: (j, 0)),
            pl.BlockSpec((BI, 1), lambda i, j: (i, 0)),
        ],
        out_specs=[
            pl.BlockSpec((BI, 2 * C), lambda i, j: (i, 0)),
            pl.BlockSpec((BI, 2 * C), lambda i, j: (i, 0)),
        ],
        out_shape=[
            jax.ShapeDtypeStruct((N, 2 * C), jnp.float32),
            jax.ShapeDtypeStruct((N, 2 * C), jnp.bfloat16),
        ],
    )

    hs = [h0]
    x = x0
    for _ in range(HOPS):
        h, x = hop_call(adj, x, norm)
        hs.append(h)

    f_gt, f_gs = pl.pallas_call(
        _tag_body,
        grid=(GI,),
        in_specs=[pl.BlockSpec((BI, 2 * C), lambda i: (i, 0))] * (HOPS + 1)
        + [
            pl.BlockSpec(((HOPS + 1) * C, C), lambda i: (0, 0)),
            pl.BlockSpec((1, C), lambda i: (0, 0)),
        ],
        out_specs=[
            pl.BlockSpec((BI, C), lambda i: (i, 0)),
            pl.BlockSpec((BI, C), lambda i: (i, 0)),
        ],
        out_shape=[
            jax.ShapeDtypeStruct((N, C), jnp.bfloat16),
            jax.ShapeDtypeStruct((N, C), jnp.bfloat16),
        ],
    )(*hs, W_tag, b_tag2)

    loss = pl.pallas_call(
        _loss_body,
        grid=(GI, GJ),
        in_specs=[
            pl.BlockSpec((BI, C), lambda i, j: (i, 0)),
            pl.BlockSpec((BJ, C), lambda i, j: (j, 0)),
        ],
        out_specs=pl.BlockSpec((1, 1), lambda i, j: (0, 0)),
        out_shape=jax.ShapeDtypeStruct((1, 1), jnp.float32),
        scratch_shapes=[
            pltpu.VMEM((BI, 1), jnp.float32),
            pltpu.VMEM((BI, 1), jnp.float32),
        ],
    )(f_gt, f_gs)

    return loss.reshape(())


# speedup vs baseline: 2.4716x; 1.8116x over previous
import jax
import jax.numpy as jnp
from jax.experimental import pallas as pl
from jax.experimental.pallas import tpu as pltpu

N = 4096
C = 128
IN = 256
HOPS = 4
TH = 0.6
T = 0.07

BI = 512
BJ = 1024
GI = N // BI
GJ = N // BJ

S_EMB = 2 * GI
S_ADJ = GI * GJ
S_HOP = HOPS * GI * GJ
S_LOSS = GI * GJ
A0 = S_EMB
H0S = A0 + S_ADJ
L0 = H0S + S_HOP
TOT = L0 + S_LOSS

BF = jnp.bfloat16
F32 = jnp.float32


def _mega_body(ft_ref, fs_ref, we_ref, be_ref, wt_ref, bt_ref, o_ref,
               h0_scr, a_scr, xb_scr, nrm_scr, dacc_scr, hacc_scr,
               tacc_scr, gt_scr, gs_scr, s_scr, p_scr):
    s = pl.program_id(0)

    @pl.when(s < A0)
    def _():
        x = jnp.where(s < GI, ft_ref[...], fs_ref[...])
        y = jnp.dot(x, we_ref[...], preferred_element_type=F32) + be_ref[...]
        y = y / jnp.sqrt(jnp.sum(y * y, axis=1, keepdims=True))
        r = jnp.where(s < GI, s, s - GI) * BI

        @pl.when(s < GI)
        def _():
            h0_scr[pl.ds(r, BI), 0:C] = y

        @pl.when(s >= GI)
        def _():
            h0_scr[pl.ds(r, BI), C:2 * C] = y

    @pl.when((s >= A0) & (s < H0S))
    def _():
        t = s - A0
        i = t // GJ
        j = t % GJ
        ei = h0_scr[pl.ds(i * BI, BI), 0:C].astype(BF)
        ej = h0_scr[pl.ds(j * BJ, BJ), 0:C].astype(BF)
        d = jax.lax.dot_general(ei, ej, (((1,), (1,)), ((), ())),
                                preferred_element_type=F32)
        base_f = jnp.where(d > TH, 1.0, 0.0)
        base = base_f.astype(BF)
        dcol = jnp.dot(base, jnp.ones((BJ, 128), BF),
                       preferred_element_type=F32)

        @pl.when(j == i // 2)
        def _():
            rowid = i * BI + jax.lax.broadcasted_iota(jnp.int32, (BI, BJ), 0)
            colid = j * BJ + jax.lax.broadcasted_iota(jnp.int32, (BI, BJ), 1)
            a_scr[pl.ds(i * BI, BI), pl.ds(j * BJ, BJ)] = (
                base_f + jnp.where(rowid == colid, 1.0, 0.0)).astype(BF)

        @pl.when(j != i // 2)
        def _():
            a_scr[pl.ds(i * BI, BI), pl.ds(j * BJ, BJ)] = base

        @pl.when(j == 0)
        def _():
            dacc_scr[...] = dcol

        @pl.when(j > 0)
        def _():
            dacc_scr[...] += dcol

        @pl.when(j == GJ - 1)
        def _():
            nv = jax.lax.rsqrt(jnp.clip(dacc_scr[:, 0:1] + 1.0, 1.0, None))
            nrm_scr[pl.ds(i * BI, BI), :] = nv
            xb_scr[0, pl.ds(i * BI, BI), :] = (
                h0_scr[pl.ds(i * BI, BI), :] * nv).astype(BF)

    @pl.when((s >= H0S) & (s < L0))
    def _():
        t = s - H0S
        k = t // (GI * GJ)
        r = t % (GI * GJ)
        i = r // GJ
        j = r % GJ
        rd = k % 2
        wr = 1 - rd
        a = a_scr[pl.ds(i * BI, BI), pl.ds(j * BJ, BJ)]
        xv = xb_scr[rd, pl.ds(j * BJ, BJ), :]
        p = jnp.dot(a, xv, preferred_element_type=F32)

        @pl.when(j == 0)
        def _():
            hacc_scr[...] = p

        @pl.when(j > 0)
        def _():
            hacc_scr[...] += p

        @pl.when(j == GJ - 1)
        def _():
            ni = nrm_scr[pl.ds(i * BI, BI), :]
            h = hacc_scr[...] * ni
            xb_scr[wr, pl.ds(i * BI, BI), :] = (h * ni).astype(BF)
            wk = wt_ref[pl.ds((k + 1) * C, C), :]
            tt = jnp.dot(h[:, 0:C], wk, preferred_element_type=F32)
            ts = jnp.dot(h[:, C:2 * C], wk, preferred_element_type=F32)

            @pl.when(k == 0)
            def _():
                h0i = h0_scr[pl.ds(i * BI, BI), :]
                w0 = wt_ref[0:C, :]
                bt = bt_ref[...]
                tacc_scr[pl.ds(i * BI, BI), 0:C] = bt + tt + jnp.dot(
                    h0i[:, 0:C], w0, preferred_element_type=F32)
                tacc_scr[pl.ds(i * BI, BI), C:2 * C] = bt + ts + jnp.dot(
                    h0i[:, C:2 * C], w0, preferred_element_type=F32)

            @pl.when(k > 0)
            def _():
                tacc_scr[pl.ds(i * BI, BI), 0:C] += tt
                tacc_scr[pl.ds(i * BI, BI), C:2 * C] += ts

            @pl.when(k == HOPS - 1)
            def _():
                rawt = tacc_scr[pl.ds(i * BI, BI), 0:C]
                raws = tacc_scr[pl.ds(i * BI, BI), C:2 * C]
                nt = jnp.sqrt(jnp.sum(rawt * rawt, axis=1, keepdims=True))
                ns = jnp.sqrt(jnp.sum(raws * raws, axis=1, keepdims=True))
                gt_scr[pl.ds(i * BI, BI), :] = (rawt / (nt * T)).astype(BF)
                gs_scr[pl.ds(i * BI, BI), :] = (raws / ns).astype(BF)

    @pl.when(s >= L0)
    def _():
        t = s - L0
        i = t // GJ
        j = t % GJ
        gt = gt_scr[pl.ds(i * BI, BI), :]
        gs = gs_scr[pl.ds(j * BJ, BJ), :]
        tile = jax.lax.dot_general(gt, gs, (((1,), (1,)), ((), ())),
                                   preferred_element_type=F32)

        @pl.when(j == i // 2)
        def _():
            rowid = i * BI + jax.lax.broadcasted_iota(jnp.int32, (BI, BJ), 0)
            colid = j * BJ + jax.lax.broadcasted_iota(jnp.int32, (BI, BJ), 1)
            dmask = rowid == colid
            p_scr[...] = jnp.sum(jnp.where(dmask, tile, 0.0), axis=1,
                                 keepdims=True)
            part = jnp.sum(jnp.where(dmask, 0.0, jnp.exp(tile)), axis=1,
                           keepdims=True)

            @pl.when(j == 0)
            def _():
                s_scr[...] = part

            @pl.when(j > 0)
            def _():
                s_scr[...] += part

        @pl.when(j != i // 2)
        def _():
            part = jnp.sum(jnp.exp(tile), axis=1, keepdims=True)

            @pl.when(j == 0)
            def _():
                s_scr[...] = part

            @pl.when(j > 0)
            def _():
                s_scr[...] += part

        @pl.when(t == 0)
        def _():
            o_ref[...] = jnp.zeros((1, 1), F32)

        @pl.when(j == GJ - 1)
        def _():
            pos = p_scr[...]
            lse = jnp.log(s_scr[...] + jnp.exp(pos))
            contrib = jnp.sum(lse - pos)
            tot = o_ref[...] + contrib
            o_ref[...] = jnp.where(i == GI - 1, tot * (1.0 / N), tot)


def kernel(feat_s, feat_t, W_embed, b_embed, W_tag, b_tag):
    loss = pl.pallas_call(
        _mega_body,
        grid=(TOT,),
        in_specs=[
            pl.BlockSpec((BI, IN), lambda s: (jnp.minimum(s, GI - 1), 0)),
            pl.BlockSpec((BI, IN),
                         lambda s: (jnp.clip(s - GI, 0, GI - 1), 0)),
            pl.BlockSpec((IN, C), lambda s: (0, 0)),
            pl.BlockSpec((1, C), lambda s: (0, 0)),
            pl.BlockSpec(((HOPS + 1) * C, C), lambda s: (0, 0)),
            pl.BlockSpec((1, C), lambda s: (0, 0)),
        ],
        out_specs=pl.BlockSpec((1, 1), lambda s: (0, 0)),
        out_shape=jax.ShapeDtypeStruct((1, 1), F32),
        scratch_shapes=[
            pltpu.VMEM((N, 2 * C), F32),
            pltpu.VMEM((N, N), BF),
            pltpu.VMEM((2, N, 2 * C), BF),
            pltpu.VMEM((N, 1), F32),
            pltpu.VMEM((BI, 128), F32),
            pltpu.VMEM((BI, 2 * C), F32),
            pltpu.VMEM((N, 2 * C), F32),
            pltpu.VMEM((N, C), BF),
            pltpu.VMEM((N, C), BF),
            pltpu.VMEM((BI, 1), F32),
            pltpu.VMEM((BI, 1), F32),
        ],
    )(feat_t, feat_s, W_embed, b_embed.reshape(1, C), W_tag,
      b_tag.reshape(1, C))
    return loss.reshape(())


# speedup vs baseline: 2.9126x; 1.1784x over previous
import jax
import jax.numpy as jnp
from jax.experimental import pallas as pl
from jax.experimental.pallas import tpu as pltpu

N = 4096
C = 128
IN = 256
HOPS = 4
TH = 0.6
T = 0.07

BI = 512
BJ = 1024
GI = N // BI
GJ = N // BJ
BH = 2048
GH = N // BH

S_EMB = 2 * GI
S_ADJ = GI * GJ
S_HOP = HOPS * GI * GH
S_LOSS = GI * GJ
A0 = S_EMB
H0S = A0 + S_ADJ
L0 = H0S + S_HOP
TOT = L0 + S_LOSS

BF = jnp.bfloat16
F32 = jnp.float32


def _mega_body(ft_ref, fs_ref, we_ref, be_ref, wt_ref, bt_ref, o_ref,
               h0_scr, a_scr, xb_scr, nrm_scr, dacc_scr, hacc_scr,
               tacc_scr, gt_scr, gs_scr, s_scr, p_scr):
    s = pl.program_id(0)

    @pl.when(s < A0)
    def _():
        x = jnp.where(s < GI, ft_ref[...], fs_ref[...])
        y = jnp.dot(x, we_ref[...], preferred_element_type=F32) + be_ref[...]
        y = y / jnp.sqrt(jnp.sum(y * y, axis=1, keepdims=True))
        r = jnp.where(s < GI, s, s - GI) * BI

        @pl.when(s < GI)
        def _():
            h0_scr[pl.ds(r, BI), 0:C] = y

        @pl.when(s >= GI)
        def _():
            h0_scr[pl.ds(r, BI), C:2 * C] = y

    @pl.when((s >= A0) & (s < H0S))
    def _():
        t = s - A0
        i = t // GJ
        j = t % GJ
        ei = h0_scr[pl.ds(i * BI, BI), 0:C].astype(BF)
        ej = h0_scr[pl.ds(j * BJ, BJ), 0:C].astype(BF)
        d = jax.lax.dot_general(ei, ej, (((1,), (1,)), ((), ())),
                                preferred_element_type=F32)
        base_f = jnp.where(d > TH, 1.0, 0.0)
        base = base_f.astype(BF)
        dcol = jnp.dot(base, jnp.ones((BJ, 128), BF),
                       preferred_element_type=F32)

        @pl.when(j == i // 2)
        def _():
            rowid = i * BI + jax.lax.broadcasted_iota(jnp.int32, (BI, BJ), 0)
            colid = j * BJ + jax.lax.broadcasted_iota(jnp.int32, (BI, BJ), 1)
            a_scr[pl.ds(i * BI, BI), pl.ds(j * BJ, BJ)] = (
                base_f + jnp.where(rowid == colid, 1.0, 0.0)).astype(BF)

        @pl.when(j != i // 2)
        def _():
            a_scr[pl.ds(i * BI, BI), pl.ds(j * BJ, BJ)] = base

        @pl.when(j == 0)
        def _():
            dacc_scr[...] = dcol

        @pl.when(j > 0)
        def _():
            dacc_scr[...] += dcol

        @pl.when(j == GJ - 1)
        def _():
            nv = jax.lax.rsqrt(jnp.clip(dacc_scr[:, 0:1] + 1.0, 1.0, None))
            nrm_scr[pl.ds(i * BI, BI), :] = nv
            xb_scr[0, pl.ds(i * BI, BI), :] = (
                h0_scr[pl.ds(i * BI, BI), :] * nv).astype(BF)

    @pl.when((s >= H0S) & (s < L0))
    def _():
        t = s - H0S
        k = t // (GI * GH)
        r = t % (GI * GH)
        i = r // GH
        j = r % GH
        rd = k % 2
        wr = 1 - rd
        a = a_scr[pl.ds(i * BI, BI), pl.ds(j * BH, BH)]
        xv = xb_scr[rd, pl.ds(j * BH, BH), :]
        p = jnp.dot(a, xv, preferred_element_type=F32)

        @pl.when(j == 0)
        def _():
            hacc_scr[...] = p

        @pl.when(j > 0)
        def _():
            hacc_scr[...] += p

        @pl.when(j == GH - 1)
        def _():
            ni = nrm_scr[pl.ds(i * BI, BI), :]
            h = hacc_scr[...] * ni
            xb_scr[wr, pl.ds(i * BI, BI), :] = (h * ni).astype(BF)
            wk = wt_ref[pl.ds((k + 1) * C, C), :]
            tt = jnp.dot(h[:, 0:C], wk, preferred_element_type=F32)
            ts = jnp.dot(h[:, C:2 * C], wk, preferred_element_type=F32)

            @pl.when(k == 0)
            def _():
                h0i = h0_scr[pl.ds(i * BI, BI), :]
                w0 = wt_ref[0:C, :]
                bt = bt_ref[...]
                tacc_scr[pl.ds(i * BI, BI), 0:C] = bt + tt + jnp.dot(
                    h0i[:, 0:C], w0, preferred_element_type=F32)
                tacc_scr[pl.ds(i * BI, BI), C:2 * C] = bt + ts + jnp.dot(
                    h0i[:, C:2 * C], w0, preferred_element_type=F32)

            @pl.when(k > 0)
            def _():
                tacc_scr[pl.ds(i * BI, BI), 0:C] += tt
                tacc_scr[pl.ds(i * BI, BI), C:2 * C] += ts

            @pl.when(k == HOPS - 1)
            def _():
                rawt = tacc_scr[pl.ds(i * BI, BI), 0:C]
                raws = tacc_scr[pl.ds(i * BI, BI), C:2 * C]
                nt = jnp.sqrt(jnp.sum(rawt * rawt, axis=1, keepdims=True))
                ns = jnp.sqrt(jnp.sum(raws * raws, axis=1, keepdims=True))
                gt_scr[pl.ds(i * BI, BI), :] = (rawt / (nt * T)).astype(BF)
                gs_scr[pl.ds(i * BI, BI), :] = (raws / ns).astype(BF)

    @pl.when(s >= L0)
    def _():
        t = s - L0
        i = t // GJ
        j = t % GJ
        gt = gt_scr[pl.ds(i * BI, BI), :]
        gs = gs_scr[pl.ds(j * BJ, BJ), :]
        tile = jax.lax.dot_general(gt, gs, (((1,), (1,)), ((), ())),
                                   preferred_element_type=F32)
        part = jnp.sum(jnp.exp(tile), axis=1, keepdims=True)

        @pl.when(j == 0)
        def _():
            s_scr[...] = part

        @pl.when(j > 0)
        def _():
            s_scr[...] += part

        @pl.when(j == i // 2)
        def _():
            rowid = i * BI + jax.lax.broadcasted_iota(jnp.int32, (BI, BJ), 0)
            colid = j * BJ + jax.lax.broadcasted_iota(jnp.int32, (BI, BJ), 1)
            p_scr[...] = jnp.sum(jnp.where(rowid == colid, tile, 0.0),
                                 axis=1, keepdims=True)

        @pl.when(t == 0)
        def _():
            o_ref[...] = jnp.zeros((1, 1), F32)

        @pl.when(j == GJ - 1)
        def _():
            pos = p_scr[...]
            lse = jnp.log(s_scr[...])
            contrib = jnp.sum(lse - pos)
            tot = o_ref[...] + contrib
            o_ref[...] = jnp.where(i == GI - 1, tot * (1.0 / N), tot)


def kernel(feat_s, feat_t, W_embed, b_embed, W_tag, b_tag):
    loss = pl.pallas_call(
        _mega_body,
        grid=(TOT,),
        in_specs=[
            pl.BlockSpec((BI, IN), lambda s: (jnp.minimum(s, GI - 1), 0)),
            pl.BlockSpec((BI, IN),
                         lambda s: (jnp.clip(s - GI, 0, GI - 1), 0)),
            pl.BlockSpec((IN, C), lambda s: (0, 0)),
            pl.BlockSpec((1, C), lambda s: (0, 0)),
            pl.BlockSpec(((HOPS + 1) * C, C), lambda s: (0, 0)),
            pl.BlockSpec((1, C), lambda s: (0, 0)),
        ],
        out_specs=pl.BlockSpec((1, 1), lambda s: (0, 0)),
        out_shape=jax.ShapeDtypeStruct((1, 1), F32),
        scratch_shapes=[
            pltpu.VMEM((N, 2 * C), F32),
            pltpu.VMEM((N, N), BF),
            pltpu.VMEM((2, N, 2 * C), BF),
            pltpu.VMEM((N, 1), F32),
            pltpu.VMEM((BI, 128), F32),
            pltpu.VMEM((BI, 2 * C), F32),
            pltpu.VMEM((N, 2 * C), F32),
            pltpu.VMEM((N, C), BF),
            pltpu.VMEM((N, C), BF),
            pltpu.VMEM((BI, 1), F32),
            pltpu.VMEM((BI, 1), F32),
        ],
    )(feat_t, feat_s, W_embed, b_embed.reshape(1, C), W_tag,
      b_tag.reshape(1, C))
    return loss.reshape(())


# speedup vs baseline: 3.1457x; 1.0800x over previous
import jax
import jax.numpy as jnp
from jax.experimental import pallas as pl
from jax.experimental.pallas import tpu as pltpu

N = 4096
C = 128
IN = 256
HOPS = 4
TH = 0.6
T = 0.07

BI = 512
BJ = 2048
GI = N // BI
GJ = N // BJ
BH = 2048
GH = N // BH

S_EMB = 2 * GI
S_ADJ = GI * GJ
S_HOP = HOPS * GI * GH
S_LOSS = GI * GJ
A0 = S_EMB
H0S = A0 + S_ADJ
L0 = H0S + S_HOP
TOT = L0 + S_LOSS

BF = jnp.bfloat16
F32 = jnp.float32


def _mega_body(ft_ref, fs_ref, we_ref, be_ref, wt_ref, bt_ref, o_ref,
               h0_scr, a_scr, xb_scr, nrm_scr, dacc_scr, hacc_scr,
               tacc_scr, gt_scr, gs_scr, s_scr, p_scr):
    s = pl.program_id(0)

    @pl.when(s < A0)
    def _():
        x = jnp.where(s < GI, ft_ref[...], fs_ref[...])
        y = jnp.dot(x, we_ref[...], preferred_element_type=F32) + be_ref[...]
        y = y / jnp.sqrt(jnp.sum(y * y, axis=1, keepdims=True))
        r = jnp.where(s < GI, s, s - GI) * BI

        @pl.when(s < GI)
        def _():
            h0_scr[pl.ds(r, BI), 0:C] = y

        @pl.when(s >= GI)
        def _():
            h0_scr[pl.ds(r, BI), C:2 * C] = y

    @pl.when((s >= A0) & (s < H0S))
    def _():
        t = s - A0
        i = t // GJ
        j = t % GJ
        ei = h0_scr[pl.ds(i * BI, BI), 0:C].astype(BF)
        ej = h0_scr[pl.ds(j * BJ, BJ), 0:C].astype(BF)
        d = jax.lax.dot_general(ei, ej, (((1,), (1,)), ((), ())),
                                preferred_element_type=F32)
        base_f = jnp.where(d > TH, 1.0, 0.0)
        base = base_f.astype(BF)
        dcol = jnp.dot(base, jnp.ones((BJ, 128), BF),
                       preferred_element_type=F32)

        @pl.when(j == (i * BI) // BJ)
        def _():
            rowid = i * BI + jax.lax.broadcasted_iota(jnp.int32, (BI, BJ), 0)
            colid = j * BJ + jax.lax.broadcasted_iota(jnp.int32, (BI, BJ), 1)
            a_scr[pl.ds(i * BI, BI), pl.ds(j * BJ, BJ)] = (
                base_f + jnp.where(rowid == colid, 1.0, 0.0)).astype(BF)

        @pl.when(j != (i * BI) // BJ)
        def _():
            a_scr[pl.ds(i * BI, BI), pl.ds(j * BJ, BJ)] = base

        @pl.when(j == 0)
        def _():
            dacc_scr[...] = dcol

        @pl.when(j > 0)
        def _():
            dacc_scr[...] += dcol

        @pl.when(j == GJ - 1)
        def _():
            nv = jax.lax.rsqrt(jnp.clip(dacc_scr[:, 0:1] + 1.0, 1.0, None))
            nrm_scr[pl.ds(i * BI, BI), :] = nv
            xb_scr[0, pl.ds(i * BI, BI), :] = (
                h0_scr[pl.ds(i * BI, BI), :] * nv).astype(BF)

    @pl.when((s >= H0S) & (s < L0))
    def _():
        t = s - H0S
        k = t // (GI * GH)
        r = t % (GI * GH)
        i = r // GH
        j = r % GH
        rd = k % 2
        wr = 1 - rd
        a = a_scr[pl.ds(i * BI, BI), pl.ds(j * BH, BH)]
        xv = xb_scr[rd, pl.ds(j * BH, BH), :]
        p = jnp.dot(a, xv, preferred_element_type=F32)

        @pl.when(j == 0)
        def _():
            hacc_scr[...] = p

        @pl.when(j > 0)
        def _():
            hacc_scr[...] += p

        @pl.when(j == GH - 1)
        def _():
            ni = nrm_scr[pl.ds(i * BI, BI), :]
            h = hacc_scr[...] * ni
            xb_scr[wr, pl.ds(i * BI, BI), :] = (h * ni).astype(BF)
            wk = wt_ref[pl.ds((k + 1) * C, C), :]
            tt = jnp.dot(h[:, 0:C], wk, preferred_element_type=F32)
            ts = jnp.dot(h[:, C:2 * C], wk, preferred_element_type=F32)

            @pl.when(k == 0)
            def _():
                h0i = h0_scr[pl.ds(i * BI, BI), :]
                w0 = wt_ref[0:C, :]
                bt = bt_ref[...]
                tacc_scr[pl.ds(i * BI, BI), 0:C] = bt + tt + jnp.dot(
                    h0i[:, 0:C], w0, preferred_element_type=F32)
                tacc_scr[pl.ds(i * BI, BI), C:2 * C] = bt + ts + jnp.dot(
                    h0i[:, C:2 * C], w0, preferred_element_type=F32)

            @pl.when(k > 0)
            def _():
                tacc_scr[pl.ds(i * BI, BI), 0:C] += tt
                tacc_scr[pl.ds(i * BI, BI), C:2 * C] += ts

            @pl.when(k == HOPS - 1)
            def _():
                rawt = tacc_scr[pl.ds(i * BI, BI), 0:C]
                raws = tacc_scr[pl.ds(i * BI, BI), C:2 * C]
                nt = jnp.sqrt(jnp.sum(rawt * rawt, axis=1, keepdims=True))
                ns = jnp.sqrt(jnp.sum(raws * raws, axis=1, keepdims=True))
                gt_scr[pl.ds(i * BI, BI), :] = (rawt / (nt * T)).astype(BF)
                gs_scr[pl.ds(i * BI, BI), :] = (raws / ns).astype(BF)

    @pl.when(s >= L0)
    def _():
        t = s - L0
        i = t // GJ
        j = t % GJ
        gt = gt_scr[pl.ds(i * BI, BI), :]
        gs = gs_scr[pl.ds(j * BJ, BJ), :]
        tile = jax.lax.dot_general(gt, gs, (((1,), (1,)), ((), ())),
                                   preferred_element_type=F32)
        part = jnp.sum(jnp.exp(tile), axis=1, keepdims=True)

        @pl.when(j == 0)
        def _():
            s_scr[...] = part

        @pl.when(j > 0)
        def _():
            s_scr[...] += part

        @pl.when(j == (i * BI) // BJ)
        def _():
            rowid = i * BI + jax.lax.broadcasted_iota(jnp.int32, (BI, BJ), 0)
            colid = j * BJ + jax.lax.broadcasted_iota(jnp.int32, (BI, BJ), 1)
            p_scr[...] = jnp.sum(jnp.where(rowid == colid, tile, 0.0),
                                 axis=1, keepdims=True)

        @pl.when(t == 0)
        def _():
            o_ref[...] = jnp.zeros((1, 1), F32)

        @pl.when(j == GJ - 1)
        def _():
            pos = p_scr[...]
            lse = jnp.log(s_scr[...])
            contrib = jnp.sum(lse - pos)
            tot = o_ref[...] + contrib
            o_ref[...] = jnp.where(i == GI - 1, tot * (1.0 / N), tot)


def kernel(feat_s, feat_t, W_embed, b_embed, W_tag, b_tag):
    loss = pl.pallas_call(
        _mega_body,
        grid=(TOT,),
        in_specs=[
            pl.BlockSpec((BI, IN), lambda s: (jnp.minimum(s, GI - 1), 0)),
            pl.BlockSpec((BI, IN),
                         lambda s: (jnp.clip(s - GI, 0, GI - 1), 0)),
            pl.BlockSpec((IN, C), lambda s: (0, 0)),
            pl.BlockSpec((1, C), lambda s: (0, 0)),
            pl.BlockSpec(((HOPS + 1) * C, C), lambda s: (0, 0)),
            pl.BlockSpec((1, C), lambda s: (0, 0)),
        ],
        out_specs=pl.BlockSpec((1, 1), lambda s: (0, 0)),
        out_shape=jax.ShapeDtypeStruct((1, 1), F32),
        scratch_shapes=[
            pltpu.VMEM((N, 2 * C), F32),
            pltpu.VMEM((N, N), BF),
            pltpu.VMEM((2, N, 2 * C), BF),
            pltpu.VMEM((N, 1), F32),
            pltpu.VMEM((BI, 128), F32),
            pltpu.VMEM((BI, 2 * C), F32),
            pltpu.VMEM((N, 2 * C), F32),
            pltpu.VMEM((N, C), BF),
            pltpu.VMEM((N, C), BF),
            pltpu.VMEM((BI, 1), F32),
            pltpu.VMEM((BI, 1), F32),
        ],
    )(feat_t, feat_s, W_embed, b_embed.reshape(1, C), W_tag,
      b_tag.reshape(1, C))
    return loss.reshape(())
